# baseline (speedup 1.0000x reference)
import functools

import jax
import jax.numpy as jnp
from jax import lax
from jax.experimental import pallas as pl
from jax.experimental.pallas import tpu as pltpu
from jax.experimental.pallas import tpu_sc as plsc

D_MODEL = 64
SCALE = 8.0
NC, NS, L = 2, 16, 16
NW = NC * NS
NBUF = 4


def _make_sc_lookup(n_rows: int, seq: int):
    mesh = plsc.VectorSubcoreMesh(core_axis_name="c", subcore_axis_name="s")
    rows_per_w = n_rows // NW
    n_groups = rows_per_w // NBUF
    splits = []
    off = 0
    while off < seq:
        n = min(128, seq - off)
        splits.append((off, n))
        off += n

    @functools.partial(
        pl.kernel,
        out_type=jax.ShapeDtypeStruct((n_rows, seq, D_MODEL), jnp.float32),
        mesh=mesh,
        scratch_types=[
            pltpu.VMEM((rows_per_w, seq), jnp.int32),
            [pltpu.VMEM((seq, D_MODEL), jnp.float32)] * NBUF,
            [pltpu.VMEM((seq, D_MODEL), jnp.float32)] * NBUF,
            [pltpu.SemaphoreType.DMA] * NBUF,
        ],
        compiler_params=pltpu.CompilerParams(use_tc_tiling_on_sc=False),
    )
    def k(x_hbm, table_hbm, out_hbm, idx_v, bufs, obufs, gsems):
        wid = lax.axis_index("s") * NC + lax.axis_index("c")
        base = wid * rows_per_w
        pltpu.sync_copy(x_hbm.at[pl.ds(base, rows_per_w)], idx_v)

        def g_start(r, b):
            for off, n in splits:
                pltpu.async_copy(
                    table_hbm.at[idx_v.at[r, pl.ds(off, n)]],
                    bufs[b].at[pl.ds(off, n)],
                    gsems[b],
                )

        def g_wait(r, b):
            for off, n in splits:
                pltpu.make_async_copy(
                    table_hbm.at[idx_v.at[r, pl.ds(off, n)]],
                    bufs[b].at[pl.ds(off, n)],
                    gsems[b],
                ).wait()

        def scale(b):
            buf, obuf = bufs[b], obufs[b]

            def srow(r, c2):
                for u in range(2):
                    for c in range(D_MODEL // L):
                        sl = pl.ds(c * L, L)
                        obuf[2 * r + u, sl] = buf[2 * r + u, sl] * SCALE
                return c2

            lax.fori_loop(0, seq // 2, srow, 0)

        def s_sync(r, b):
            pltpu.sync_copy(obufs[b], out_hbm.at[base + r])

        for b in range(NBUF):
            g_start(b, b)

        def step(g, carry):
            for b in range(NBUF):
                r = g * NBUF + b
                g_wait(r, b)
                scale(b)
                g_start(r + NBUF, b)
                s_sync(r, b)
            return carry

        lax.fori_loop(0, n_groups - 1, step, 0)

        for b in range(NBUF):
            r = (n_groups - 1) * NBUF + b
            g_wait(r, b)
            scale(b)
            s_sync(r, b)

    return k


def kernel(x, table):
    n_rows, seq = x.shape
    out = _make_sc_lookup(n_rows, seq)(x.astype(jnp.int32), table)
    return out


# speedup vs baseline: 1.0334x; 1.0334x over previous
import functools

import jax
import jax.numpy as jnp
from jax import lax
from jax.experimental import pallas as pl
from jax.experimental.pallas import tpu as pltpu
from jax.experimental.pallas import tpu_sc as plsc

D_MODEL = 64
SCALE = 8.0
NC, NS, L = 2, 16, 16
NW = NC * NS
CH = 128
NBUF = 4


def _make_sc_lookup(seq: int, n_rows: int):
    mesh = plsc.VectorSubcoreMesh(core_axis_name="c", subcore_axis_name="s")
    n_groups = seq // NBUF

    @functools.partial(
        pl.kernel,
        out_type=jax.ShapeDtypeStruct((seq, n_rows, D_MODEL), jnp.float32),
        mesh=mesh,
        scratch_types=[
            pltpu.VMEM((seq, CH), jnp.int32),
            [pltpu.VMEM((CH, D_MODEL), jnp.float32)] * NBUF,
            [pltpu.VMEM((CH, D_MODEL), jnp.float32)] * NBUF,
            [pltpu.SemaphoreType.DMA] * NBUF,
        ],
        compiler_params=pltpu.CompilerParams(use_tc_tiling_on_sc=False),
    )
    def k(xt_hbm, table_hbm, out_hbm, idx_v, bufs, obufs, gsems):
        wid = lax.axis_index("s") * NC + lax.axis_index("c")
        col = wid * CH
        pltpu.sync_copy(xt_hbm.at[:, pl.ds(col, CH)], idx_v)

        def g_start(t, b):
            pltpu.async_copy(table_hbm.at[idx_v.at[t]], bufs[b], gsems[b])

        def g_wait(t, b):
            pltpu.make_async_copy(
                table_hbm.at[idx_v.at[t]], bufs[b], gsems[b]
            ).wait()

        def scale(b):
            buf, obuf = bufs[b], obufs[b]

            def srow(r, c2):
                for u in range(2):
                    for c in range(D_MODEL // L):
                        sl = pl.ds(c * L, L)
                        obuf[2 * r + u, sl] = buf[2 * r + u, sl] * SCALE
                return c2

            lax.fori_loop(0, CH // 2, srow, 0)

        def s_sync(t, b):
            pltpu.sync_copy(obufs[b], out_hbm.at[t, pl.ds(col, CH)])

        for b in range(NBUF):
            g_start(b, b)

        def step(g, carry):
            for b in range(NBUF):
                t = g * NBUF + b
                g_wait(t, b)
                scale(b)
                g_start(t + NBUF, b)
                s_sync(t, b)
            return carry

        lax.fori_loop(0, n_groups - 1, step, 0)

        for b in range(NBUF):
            t = (n_groups - 1) * NBUF + b
            g_wait(t, b)
            scale(b)
            s_sync(t, b)

    return k


def kernel(x, table):
    n_rows, seq = x.shape
    outp = _make_sc_lookup(seq, n_rows)(x.T.astype(jnp.int32), table)
    return outp.transpose(1, 0, 2)
